# baseline (speedup 1.0000x reference)
import jax
import jax.numpy as jnp
from jax import lax
from jax.experimental import pallas as pl
from jax.experimental.pallas import tpu as pltpu
from jax.experimental.pallas import tpu_sc as plsc


B, L, V, D, NDOCS = 4096, 32, 30522, 768, 10
NEMBS = NDOCS + 1
NW = 32
RPW = B // NW
VPAD = 30528
D2 = D // 2
DCW = D2 // 16
BLK = 256
DBLK = 512
VBLK = 1272


def _softmax16(ew):
    e = jnp.exp(ew - jnp.max(ew))
    return e / jnp.sum(e)


def _pack_body(x_ref, o_ref):
    x = x_ref[...]
    a = x[:, :D2].astype(jnp.bfloat16)
    b = x[:, D2:].astype(jnp.bfloat16)
    au = lax.bitcast_convert_type(a, jnp.uint16).astype(jnp.uint32)
    bu = lax.bitcast_convert_type(b, jnp.uint16).astype(jnp.uint32)
    o_ref[...] = lax.bitcast_convert_type(au | (bu << 16), jnp.float32)


def _doc_body(ew_ref, d_ref, o_ref):
    ws = _softmax16(ew_ref[...])
    acc = ws[0:1, 1:2] * d_ref[:, 0:D]
    for k in range(1, NDOCS):
        acc = acc + ws[0:1, k + 1:k + 2] * d_ref[:, k * D:(k + 1) * D]
    o_ref[...] = acc


def _combine_body(ew_ref, q_ref, dp_ref, o_ref):
    ws = _softmax16(ew_ref[...])
    u = lax.bitcast_convert_type(q_ref[...], jnp.uint32)
    lowf = lax.bitcast_convert_type(u << 16, jnp.float32)
    highf = lax.bitcast_convert_type(u & jnp.uint32(0xFFFF0000), jnp.float32)
    q1 = jnp.concatenate([lowf, highf], axis=1)
    o_ref[...] = ws[0:1, 0:1] * q1 + dp_ref[...]


def _sc_body(ids_hbm, table_hbm, tokw_hbm, out_hbm,
             tokw_v, ids_v,
             rows_v0, rows_v1, acc_v0, acc_v1,
             rsem0, rsem1, osem0, osem1):
    cid = lax.axis_index("c")
    sid = lax.axis_index("s")
    wid = sid * 2 + cid
    base = wid * RPW

    pltpu.sync_copy(tokw_hbm, tokw_v)
    pltpu.sync_copy(ids_hbm.at[pl.ds(base * L, RPW * L)], ids_v)

    def start_row(r, rowsbuf, rsem):
        off = pl.multiple_of(r * L, L)
        pltpu.make_async_copy(
            table_hbm.at[ids_v.at[pl.ds(off, L)]], rowsbuf, rsem).start()

    def wait_row(r, rowsbuf, rsem):
        off = pl.multiple_of(r * L, L)
        pltpu.make_async_copy(
            table_hbm.at[ids_v.at[pl.ds(off, L)]], rowsbuf, rsem).wait()

    def compute_row(r, rowsbuf, accbuf, osem, not_first):
        off = pl.multiple_of(r * L, L)
        i1 = ids_v[pl.ds(off, 16)]
        i2 = ids_v[pl.ds(off + 16, 16)]
        w1 = plsc.load_gather(tokw_v, [i1])
        w2 = plsc.load_gather(tokw_v, [i2])
        m = jnp.full((16,), jnp.max(jnp.maximum(w1, w2)))
        e1 = jnp.exp(w1 - m)
        e2 = jnp.exp(w2 - m)
        coef = jnp.full((16,), 1.0) / jnp.full((16,), jnp.sum(e1 + e2))
        e1 = e1 * coef
        e2 = e2 * coef
        els = [e1[i] for i in range(16)] + [e2[i] for i in range(16)]
        elbs = []
        for el in els:
            v = jnp.full((16,), el)
            elbs.append(plsc.pack(v, v, format=plsc.PackFormat.INTERLEAVED))

        @pl.when(not_first)
        def _():
            pltpu.make_async_copy(
                accbuf, out_hbm.at[pl.ds((base + r) * D2, D2)], osem).wait()

        def chunk_step(c, carry):
            cs = pl.multiple_of(c * 16, 16)
            acc = elbs[0] * plsc.bitcast(rowsbuf[0, pl.ds(cs, 16)],
                                         jnp.bfloat16)
            for l in range(1, L):
                acc = acc + elbs[l] * plsc.bitcast(
                    rowsbuf[l, pl.ds(cs, 16)], jnp.bfloat16)
            accbuf[pl.ds(cs, 16)] = plsc.bitcast(acc, jnp.float32)
            return carry
        lax.fori_loop(0, DCW, chunk_step, 0)

        pltpu.make_async_copy(
            accbuf, out_hbm.at[pl.ds((base + r) * D2, D2)], osem).start()

    half = RPW // 2
    start_row(0, rows_v0, rsem0)

    def outer(rr, carry):
        r0 = rr * 2
        start_row(r0 + 1, rows_v1, rsem1)
        wait_row(r0, rows_v0, rsem0)
        compute_row(r0, rows_v0, acc_v0, osem0, rr > 0)

        @pl.when(rr + 1 < half)
        def _():
            start_row(r0 + 2, rows_v0, rsem0)
        wait_row(r0 + 1, rows_v1, rsem1)
        compute_row(r0 + 1, rows_v1, acc_v1, osem1, rr > 0)
        return carry
    lax.fori_loop(0, half, outer, 0)

    pltpu.make_async_copy(
        acc_v0, out_hbm.at[pl.ds((base + RPW - 2) * D2, D2)], osem0).wait()
    pltpu.make_async_copy(
        acc_v1, out_hbm.at[pl.ds((base + RPW - 1) * D2, D2)], osem1).wait()


@jax.jit
def _run(ids_flat, d_embs, tok_embs, tokw_pad, ew_pad):
    ew16 = ew_pad.reshape(1, 16)

    table32 = pl.pallas_call(
        _pack_body,
        grid=(VPAD // VBLK,),
        in_specs=[pl.BlockSpec((VBLK, D), lambda i: (i, 0))],
        out_specs=pl.BlockSpec((VBLK, D2), lambda i: (i, 0)),
        out_shape=jax.ShapeDtypeStruct((VPAD, D2), jnp.float32),
    )(tok_embs)

    mesh = plsc.VectorSubcoreMesh(core_axis_name="c", subcore_axis_name="s")
    f = pl.kernel(
        _sc_body,
        out_type=jax.ShapeDtypeStruct((B * D2,), jnp.float32),
        mesh=mesh,
        compiler_params=pltpu.CompilerParams(needs_layout_passes=False),
        scratch_types=[
            pltpu.VMEM((VPAD,), jnp.float32),
            pltpu.VMEM((RPW * L,), jnp.int32),
            pltpu.VMEM((L, D2), jnp.float32),
            pltpu.VMEM((L, D2), jnp.float32),
            pltpu.VMEM((D2,), jnp.float32),
            pltpu.VMEM((D2,), jnp.float32),
            pltpu.SemaphoreType.DMA,
            pltpu.SemaphoreType.DMA,
            pltpu.SemaphoreType.DMA,
            pltpu.SemaphoreType.DMA,
        ],
    )
    q1words = f(ids_flat, table32, tokw_pad)

    docpart = pl.pallas_call(
        _doc_body,
        grid=(B // DBLK,),
        in_specs=[
            pl.BlockSpec((1, 16), lambda i: (0, 0)),
            pl.BlockSpec((DBLK, NDOCS * D), lambda i: (i, 0)),
        ],
        out_specs=pl.BlockSpec((DBLK, D), lambda i: (i, 0)),
        out_shape=jax.ShapeDtypeStruct((B, D), jnp.float32),
    )(ew16, d_embs.reshape(B, NDOCS * D))

    out = pl.pallas_call(
        _combine_body,
        grid=(B // BLK,),
        in_specs=[
            pl.BlockSpec((1, 16), lambda i: (0, 0)),
            pl.BlockSpec((BLK, D2), lambda i: (i, 0)),
            pl.BlockSpec((BLK, D), lambda i: (i, 0)),
        ],
        out_specs=pl.BlockSpec((BLK, D), lambda i: (i, 0)),
        out_shape=jax.ShapeDtypeStruct((B, D), jnp.float32),
    )(ew16, q1words.reshape(B, D2), docpart)
    return out


def kernel(input_ids, attention_mask, d_embs, tok_embs, tok_embs_avg_weights,
           embs_avg_weights):
    del attention_mask
    ids_flat = input_ids.reshape(B * L)
    tokw_pad = jnp.pad(tok_embs_avg_weights, (0, VPAD - V))
    ew_pad = jnp.pad(embs_avg_weights, (0, 16 - NEMBS), constant_values=-1e30)
    return _run(ids_flat, d_embs, tok_embs, tokw_pad, ew_pad)


# speedup vs baseline: 1.1964x; 1.1964x over previous
import jax
import jax.numpy as jnp
from jax import lax
from jax.experimental import pallas as pl
from jax.experimental.pallas import tpu as pltpu
from jax.experimental.pallas import tpu_sc as plsc


B, L, V, D, NDOCS = 4096, 32, 30522, 768, 10
NEMBS = NDOCS + 1
NW = 32
RPW = B // NW
VPAD = 30528
D2 = D // 2
DCW = D2 // 16
BLK = 256
DBLK = 512
VBLK = 1272


def _softmax16(ew):
    e = jnp.exp(ew - jnp.max(ew))
    return e / jnp.sum(e)


def _pack_body(x_ref, o_ref):
    x = x_ref[...]
    a = x[:, :D2].astype(jnp.bfloat16)
    b = x[:, D2:].astype(jnp.bfloat16)
    au = lax.bitcast_convert_type(a, jnp.uint16).astype(jnp.uint32)
    bu = lax.bitcast_convert_type(b, jnp.uint16).astype(jnp.uint32)
    o_ref[...] = lax.bitcast_convert_type(au | (bu << 16), jnp.float32)


def _doc_body(ew_ref, d_ref, o_ref):
    ws = _softmax16(ew_ref[...])
    acc = ws[0:1, 1:2] * d_ref[:, 0:D].astype(jnp.float32)
    for k in range(1, NDOCS):
        acc = acc + ws[0:1, k + 1:k + 2] * (
            d_ref[:, k * D:(k + 1) * D].astype(jnp.float32))
    o_ref[...] = acc


def _combine_body(ew_ref, q_ref, dp_ref, o_ref):
    ws = _softmax16(ew_ref[...])
    u = lax.bitcast_convert_type(q_ref[...], jnp.uint32)
    lowf = lax.bitcast_convert_type(u << 16, jnp.float32)
    highf = lax.bitcast_convert_type(u & jnp.uint32(0xFFFF0000), jnp.float32)
    q1 = jnp.concatenate([lowf, highf], axis=1)
    o_ref[...] = ws[0:1, 0:1] * q1 + dp_ref[...]


def _sc_body(ids_hbm, table_hbm, tokw_hbm, out_hbm,
             tokw_v, ids_v,
             rows_v0, rows_v1, acc_v0, acc_v1,
             rsem0, rsem1, osem0, osem1):
    cid = lax.axis_index("c")
    sid = lax.axis_index("s")
    wid = sid * 2 + cid
    base = wid * RPW

    pltpu.sync_copy(tokw_hbm, tokw_v)
    pltpu.sync_copy(ids_hbm.at[pl.ds(base * L, RPW * L)], ids_v)

    def start_row(r, rowsbuf, rsem):
        off = pl.multiple_of(r * L, L)
        pltpu.make_async_copy(
            table_hbm.at[ids_v.at[pl.ds(off, L)]], rowsbuf, rsem).start()

    def wait_row(r, rowsbuf, rsem):
        off = pl.multiple_of(r * L, L)
        pltpu.make_async_copy(
            table_hbm.at[ids_v.at[pl.ds(off, L)]], rowsbuf, rsem).wait()

    def compute_row(r, rowsbuf, accbuf, osem, not_first):
        off = pl.multiple_of(r * L, L)
        i1 = ids_v[pl.ds(off, 16)]
        i2 = ids_v[pl.ds(off + 16, 16)]
        w1 = plsc.load_gather(tokw_v, [i1])
        w2 = plsc.load_gather(tokw_v, [i2])
        m = jnp.full((16,), jnp.max(jnp.maximum(w1, w2)))
        e1 = jnp.exp(w1 - m)
        e2 = jnp.exp(w2 - m)
        coef = jnp.full((16,), 1.0) / jnp.full((16,), jnp.sum(e1 + e2))
        e1 = e1 * coef
        e2 = e2 * coef
        els = [e1[i] for i in range(16)] + [e2[i] for i in range(16)]
        elbs = []
        for el in els:
            v = jnp.full((16,), el)
            elbs.append(plsc.pack(v, v, format=plsc.PackFormat.INTERLEAVED))

        @pl.when(not_first)
        def _():
            pltpu.make_async_copy(
                accbuf, out_hbm.at[pl.ds((base + r) * D2, D2)], osem).wait()

        def chunk_step(c, carry):
            cs = pl.multiple_of(c * 16, 16)
            acc = elbs[0] * plsc.bitcast(rowsbuf[0, pl.ds(cs, 16)],
                                         jnp.bfloat16)
            for l in range(1, L):
                acc = acc + elbs[l] * plsc.bitcast(
                    rowsbuf[l, pl.ds(cs, 16)], jnp.bfloat16)
            accbuf[pl.ds(cs, 16)] = plsc.bitcast(acc, jnp.float32)
            return carry
        lax.fori_loop(0, DCW, chunk_step, 0)

        pltpu.make_async_copy(
            accbuf, out_hbm.at[pl.ds((base + r) * D2, D2)], osem).start()

    half = RPW // 2
    start_row(0, rows_v0, rsem0)

    def outer(rr, carry):
        r0 = rr * 2
        start_row(r0 + 1, rows_v1, rsem1)
        wait_row(r0, rows_v0, rsem0)
        compute_row(r0, rows_v0, acc_v0, osem0, rr > 0)

        @pl.when(rr + 1 < half)
        def _():
            start_row(r0 + 2, rows_v0, rsem0)
        wait_row(r0 + 1, rows_v1, rsem1)
        compute_row(r0 + 1, rows_v1, acc_v1, osem1, rr > 0)
        return carry
    lax.fori_loop(0, half, outer, 0)

    pltpu.make_async_copy(
        acc_v0, out_hbm.at[pl.ds((base + RPW - 2) * D2, D2)], osem0).wait()
    pltpu.make_async_copy(
        acc_v1, out_hbm.at[pl.ds((base + RPW - 1) * D2, D2)], osem1).wait()


@jax.jit
def _run(ids_flat, d_embs, tok_embs, tokw_pad, ew_pad):
    ew16 = ew_pad.reshape(1, 16)

    table32 = pl.pallas_call(
        _pack_body,
        grid=(VPAD // VBLK,),
        in_specs=[pl.BlockSpec((VBLK, D), lambda i: (i, 0))],
        out_specs=pl.BlockSpec((VBLK, D2), lambda i: (i, 0)),
        out_shape=jax.ShapeDtypeStruct((VPAD, D2), jnp.float32),
    )(tok_embs)

    mesh = plsc.VectorSubcoreMesh(core_axis_name="c", subcore_axis_name="s")
    f = pl.kernel(
        _sc_body,
        out_type=jax.ShapeDtypeStruct((B * D2,), jnp.float32),
        mesh=mesh,
        compiler_params=pltpu.CompilerParams(needs_layout_passes=False),
        scratch_types=[
            pltpu.VMEM((VPAD,), jnp.float32),
            pltpu.VMEM((RPW * L,), jnp.int32),
            pltpu.VMEM((L, D2), jnp.float32),
            pltpu.VMEM((L, D2), jnp.float32),
            pltpu.VMEM((D2,), jnp.float32),
            pltpu.VMEM((D2,), jnp.float32),
            pltpu.SemaphoreType.DMA,
            pltpu.SemaphoreType.DMA,
            pltpu.SemaphoreType.DMA,
            pltpu.SemaphoreType.DMA,
        ],
    )
    q1words = f(ids_flat, table32, tokw_pad)

    docpart = pl.pallas_call(
        _doc_body,
        grid=(B // DBLK,),
        in_specs=[
            pl.BlockSpec((1, 16), lambda i: (0, 0)),
            pl.BlockSpec((DBLK, NDOCS * D), lambda i: (i, 0)),
        ],
        out_specs=pl.BlockSpec((DBLK, D), lambda i: (i, 0)),
        out_shape=jax.ShapeDtypeStruct((B, D), jnp.float32),
    )(ew16, d_embs.astype(jnp.bfloat16).reshape(B, NDOCS * D))

    out = pl.pallas_call(
        _combine_body,
        grid=(B // BLK,),
        in_specs=[
            pl.BlockSpec((1, 16), lambda i: (0, 0)),
            pl.BlockSpec((BLK, D2), lambda i: (i, 0)),
            pl.BlockSpec((BLK, D), lambda i: (i, 0)),
        ],
        out_specs=pl.BlockSpec((BLK, D), lambda i: (i, 0)),
        out_shape=jax.ShapeDtypeStruct((B, D), jnp.float32),
    )(ew16, q1words.reshape(B, D2), docpart)
    return out


def kernel(input_ids, attention_mask, d_embs, tok_embs, tok_embs_avg_weights,
           embs_avg_weights):
    del attention_mask
    ids_flat = input_ids.reshape(B * L)
    tokw_pad = jnp.pad(tok_embs_avg_weights, (0, VPAD - V))
    ew_pad = jnp.pad(embs_avg_weights, (0, 16 - NEMBS), constant_values=-1e30)
    return _run(ids_flat, d_embs, tok_embs, tokw_pad, ew_pad)


# speedup vs baseline: 1.1993x; 1.0025x over previous
import jax
import jax.numpy as jnp
from jax import lax
from jax.experimental import pallas as pl
from jax.experimental.pallas import tpu as pltpu
from jax.experimental.pallas import tpu_sc as plsc


B, L, V, D, NDOCS = 4096, 32, 30522, 768, 10
NEMBS = NDOCS + 1
NW = 32
RPW = B // NW
VPAD = 30528
D2 = D // 2
DCW = D2 // 16
BLK = 256
DBLK = 512
VBLK = 1272


def _softmax16(ew):
    e = jnp.exp(ew - jnp.max(ew))
    return e / jnp.sum(e)


def _pack_body(x_ref, o_ref):
    x = x_ref[...]
    a = x[:, :D2].astype(jnp.bfloat16)
    b = x[:, D2:].astype(jnp.bfloat16)
    au = lax.bitcast_convert_type(a, jnp.uint16).astype(jnp.uint32)
    bu = lax.bitcast_convert_type(b, jnp.uint16).astype(jnp.uint32)
    o_ref[...] = lax.bitcast_convert_type(au | (bu << 16), jnp.float32)


def _doc_body(ew_ref, d_ref, o_ref):
    ws = _softmax16(ew_ref[...])
    acc = ws[0:1, 1:2] * d_ref[:, 0:D].astype(jnp.float32)
    for k in range(1, NDOCS):
        acc = acc + ws[0:1, k + 1:k + 2] * (
            d_ref[:, k * D:(k + 1) * D].astype(jnp.float32))
    o_ref[...] = acc


def _combine_body(ew_ref, q_ref, dp_ref, o_ref):
    ws = _softmax16(ew_ref[...])
    u = lax.bitcast_convert_type(q_ref[...], jnp.uint32)
    lowf = lax.bitcast_convert_type(u << 16, jnp.float32)
    highf = lax.bitcast_convert_type(u & jnp.uint32(0xFFFF0000), jnp.float32)
    q1 = jnp.concatenate([lowf, highf], axis=1)
    o_ref[...] = ws[0:1, 0:1] * q1 + dp_ref[...]


def _sc_body(ids_hbm, table_hbm, tokw_hbm, out_hbm,
             tokw_v, ids_v,
             rows_v0, rows_v1, acc_v0, acc_v1,
             rsem0, rsem1, osem0, osem1):
    cid = lax.axis_index("c")
    sid = lax.axis_index("s")
    wid = sid * 2 + cid
    base = wid * RPW

    pltpu.sync_copy(tokw_hbm, tokw_v)
    pltpu.sync_copy(ids_hbm.at[pl.ds(base * L, RPW * L)], ids_v)

    def start_row(r, rowsbuf, rsem):
        off = pl.multiple_of(r * L, L)
        pltpu.make_async_copy(
            table_hbm.at[ids_v.at[pl.ds(off, L)]], rowsbuf, rsem).start()

    def wait_row(r, rowsbuf, rsem):
        off = pl.multiple_of(r * L, L)
        pltpu.make_async_copy(
            table_hbm.at[ids_v.at[pl.ds(off, L)]], rowsbuf, rsem).wait()

    def compute_row(r, rowsbuf, accbuf, osem, not_first):
        off = pl.multiple_of(r * L, L)
        i1 = ids_v[pl.ds(off, 16)]
        i2 = ids_v[pl.ds(off + 16, 16)]
        w1 = plsc.load_gather(tokw_v, [i1])
        w2 = plsc.load_gather(tokw_v, [i2])
        m = jnp.full((16,), jnp.max(jnp.maximum(w1, w2)))
        e1 = jnp.exp(w1 - m)
        e2 = jnp.exp(w2 - m)
        coef = jnp.full((16,), 1.0) / jnp.full((16,), jnp.sum(e1 + e2))
        e1 = e1 * coef
        e2 = e2 * coef
        els = [e1[i] for i in range(16)] + [e2[i] for i in range(16)]
        elbs = []
        for el in els:
            v = jnp.full((16,), el)
            elbs.append(plsc.pack(v, v, format=plsc.PackFormat.INTERLEAVED))

        @pl.when(not_first)
        def _():
            pltpu.make_async_copy(
                accbuf, out_hbm.at[pl.ds((base + r) * D2, D2)], osem).wait()

        def chunk_step(c, carry):
            cs = pl.multiple_of(c * 16, 16)
            acc = elbs[0] * plsc.bitcast(rowsbuf[0, pl.ds(cs, 16)],
                                         jnp.bfloat16)
            for l in range(1, L):
                acc = acc + elbs[l] * plsc.bitcast(
                    rowsbuf[l, pl.ds(cs, 16)], jnp.bfloat16)
            accbuf[pl.ds(cs, 16)] = plsc.bitcast(acc, jnp.float32)
            return carry
        lax.fori_loop(0, DCW, chunk_step, 0)

        pltpu.make_async_copy(
            accbuf, out_hbm.at[pl.ds((base + r) * D2, D2)], osem).start()

    half = RPW // 2
    start_row(0, rows_v0, rsem0)

    def outer(rr, carry):
        r0 = rr * 2
        start_row(r0 + 1, rows_v1, rsem1)
        wait_row(r0, rows_v0, rsem0)
        compute_row(r0, rows_v0, acc_v0, osem0, rr > 0)

        @pl.when(rr + 1 < half)
        def _():
            start_row(r0 + 2, rows_v0, rsem0)
        wait_row(r0 + 1, rows_v1, rsem1)
        compute_row(r0 + 1, rows_v1, acc_v1, osem1, rr > 0)
        return carry
    lax.fori_loop(0, half, outer, 0)

    pltpu.make_async_copy(
        acc_v0, out_hbm.at[pl.ds((base + RPW - 2) * D2, D2)], osem0).wait()
    pltpu.make_async_copy(
        acc_v1, out_hbm.at[pl.ds((base + RPW - 1) * D2, D2)], osem1).wait()


@jax.jit
def _run(ids_flat, d_embs, tok_embs, tokw_pad, ew_pad):
    ew16 = ew_pad.reshape(1, 16)

    table32 = pl.pallas_call(
        _pack_body,
        grid=(VPAD // VBLK,),
        in_specs=[pl.BlockSpec((VBLK, D), lambda i: (i, 0))],
        out_specs=pl.BlockSpec((VBLK, D2), lambda i: (i, 0)),
        out_shape=jax.ShapeDtypeStruct((VPAD, D2), jnp.float32),
    )(tok_embs)

    mesh = plsc.VectorSubcoreMesh(core_axis_name="c", subcore_axis_name="s")
    f = pl.kernel(
        _sc_body,
        out_type=jax.ShapeDtypeStruct((B * D2,), jnp.float32),
        mesh=mesh,
        compiler_params=pltpu.CompilerParams(needs_layout_passes=False),
        scratch_types=[
            pltpu.VMEM((VPAD,), jnp.float32),
            pltpu.VMEM((RPW * L,), jnp.int32),
            pltpu.VMEM((L, D2), jnp.float32),
            pltpu.VMEM((L, D2), jnp.float32),
            pltpu.VMEM((D2,), jnp.float32),
            pltpu.VMEM((D2,), jnp.float32),
            pltpu.SemaphoreType.DMA,
            pltpu.SemaphoreType.DMA,
            pltpu.SemaphoreType.DMA,
            pltpu.SemaphoreType.DMA,
        ],
    )
    q1words = f(ids_flat, table32, tokw_pad)

    docpart = pl.pallas_call(
        _doc_body,
        grid=(B // DBLK,),
        in_specs=[
            pl.BlockSpec((1, 16), lambda i: (0, 0)),
            pl.BlockSpec((DBLK, NDOCS * D), lambda i: (i, 0)),
        ],
        out_specs=pl.BlockSpec((DBLK, D), lambda i: (i, 0)),
        out_shape=jax.ShapeDtypeStruct((B, D), jnp.float32),
    )(ew16, d_embs.reshape(B, NDOCS * D).astype(jnp.bfloat16))

    out = pl.pallas_call(
        _combine_body,
        grid=(B // BLK,),
        in_specs=[
            pl.BlockSpec((1, 16), lambda i: (0, 0)),
            pl.BlockSpec((BLK, D2), lambda i: (i, 0)),
            pl.BlockSpec((BLK, D), lambda i: (i, 0)),
        ],
        out_specs=pl.BlockSpec((BLK, D), lambda i: (i, 0)),
        out_shape=jax.ShapeDtypeStruct((B, D), jnp.float32),
    )(ew16, q1words.reshape(B, D2), docpart)
    return out


def kernel(input_ids, attention_mask, d_embs, tok_embs, tok_embs_avg_weights,
           embs_avg_weights):
    del attention_mask
    ids_flat = input_ids.reshape(B * L)
    tokw_pad = jnp.pad(tok_embs_avg_weights, (0, VPAD - V))
    ew_pad = jnp.pad(embs_avg_weights, (0, 16 - NEMBS), constant_values=-1e30)
    return _run(ids_flat, d_embs, tok_embs, tokw_pad, ew_pad)
